# baseline (speedup 1.0000x reference)
import functools

import jax
import jax.numpy as jnp
from jax import lax
from jax.experimental import pallas as pl
from jax.experimental.pallas import tpu as pltpu
from jax.experimental.pallas import tpu_sc as plsc

D = 128
GATHER = 128
CHUNK = 128
NBUF = 7
LEAD = 6
NC, NS = 2, 16
NW = NC * NS


def _emb_body(idx_hbm, table_hbm, out_hbm, idx_v, bufs, gsem, ssem):
    n = idx_hbm.shape[0] // (NW * CHUNK)
    wid = lax.axis_index("s") * NC + lax.axis_index("c")
    base = wid * n
    pltpu.sync_copy(idx_hbm.at[pl.ds(base * CHUNK, n * CHUNK)], idx_v)

    def gather(j, b):
        for g in range(CHUNK // GATHER):
            pltpu.async_copy(
                table_hbm.at[idx_v.at[pl.ds(j * CHUNK + g * GATHER, GATHER)]],
                bufs.at[b].at[pl.ds(g * GATHER, GATHER)], gsem.at[b])

    def wait_g(b):
        pltpu.make_async_copy(
            table_hbm.at[pl.ds(0, CHUNK)], bufs.at[b], gsem.at[b]).wait()

    def scatter(j, b):
        pltpu.async_copy(
            bufs.at[b], out_hbm.at[pl.ds((base + j) * CHUNK, CHUNK)],
            ssem.at[b])

    def wait_s(b):
        pltpu.make_async_copy(
            bufs.at[b], out_hbm.at[pl.ds(0, CHUNK)], ssem.at[b]).wait()

    def visit(j, b, refill, swait):
        wait_g(b)
        scatter(j, b)
        if refill:
            bp = (b + LEAD) % NBUF
            if swait:
                wait_s(bp)
            gather(j + LEAD, bp)

    for c in range(LEAD):
        gather(c, c)
    for j in range(NBUF):
        visit(j, j, refill=True, swait=j >= NBUF - LEAD)

    def body(i, carry):
        for b in range(NBUF):
            visit(NBUF * i + b, b, refill=True, swait=True)
        return carry

    lax.fori_loop(1, (n - NBUF - n % NBUF) // NBUF, body, 0)

    for j in range(n - NBUF - n % NBUF, n):
        visit(j, j % NBUF, refill=j + LEAD < n, swait=True)
    for b in range(NBUF):
        wait_s(b)


@functools.partial(jax.jit, static_argnames=("total_rows",))
def _emb_call(idx1d, weight, total_rows):
    mesh = plsc.VectorSubcoreMesh(core_axis_name="c", subcore_axis_name="s")
    f = pl.kernel(
        _emb_body,
        out_type=jax.ShapeDtypeStruct((total_rows, D), jnp.float32),
        mesh=mesh,
        compiler_params=pltpu.CompilerParams(use_tc_tiling_on_sc=False),
        scratch_types=[
            pltpu.VMEM((total_rows // NW,), jnp.int32),
            pltpu.VMEM((NBUF, CHUNK, D), jnp.float32),
            pltpu.SemaphoreType.DMA((NBUF,)),
            pltpu.SemaphoreType.DMA((NBUF,)),
        ],
    )
    return f(idx1d, weight)


def kernel(token_ids, weight):
    b, s = token_ids.shape
    total = b * s
    idx1d = token_ids.astype(jnp.int32).T.reshape(total)
    out = _emb_call(idx1d, weight, total)
    return out.reshape(s, b, D).transpose(1, 0, 2)


# speedup vs baseline: 1.0094x; 1.0094x over previous
import functools

import jax
import jax.numpy as jnp
from jax import lax
from jax.experimental import pallas as pl
from jax.experimental.pallas import tpu as pltpu
from jax.experimental.pallas import tpu_sc as plsc

D = 128
GATHER = 128
CHUNK = 128
NBUF = 7
LEAD = 6
NC, NS = 2, 16
NW = NC * NS


def _emb_body(idx_hbm, table_hbm, out_hbm, idx_v, bufs, gsem, ssem):
    n = idx_hbm.shape[0] // (NW * CHUNK)
    wid = lax.axis_index("s") * NC + lax.axis_index("c")
    base = wid * n
    pltpu.sync_copy(idx_hbm.at[pl.ds(base * CHUNK, n * CHUNK)], idx_v)

    def gather(j, b):
        for g in range(CHUNK // GATHER):
            pltpu.async_copy(
                table_hbm.at[idx_v.at[pl.ds(j * CHUNK + g * GATHER, GATHER)]],
                bufs.at[b].at[pl.ds(g * GATHER, GATHER)], gsem.at[b])

    def wait_g(b):
        pltpu.make_async_copy(
            table_hbm.at[pl.ds(0, CHUNK)], bufs.at[b], gsem.at[b]).wait()

    def scatter(j, b):
        pltpu.async_copy(
            bufs.at[b], out_hbm.at[pl.ds((base + j) * CHUNK, CHUNK)],
            ssem.at[b])

    def wait_s(b):
        pltpu.make_async_copy(
            bufs.at[b], out_hbm.at[pl.ds(0, CHUNK)], ssem.at[b]).wait()

    def visit(j, b, refill, swait):
        wait_g(b)
        scatter(j, b)
        if refill:
            bp = (b + LEAD) % NBUF
            if swait:
                wait_s(bp)
            gather(j + LEAD, bp)

    for c in range(LEAD):
        gather(c, c)
    for j in range(NBUF):
        visit(j, j, refill=True, swait=j >= NBUF - LEAD)

    def body(i, carry):
        for b in range(NBUF):
            visit(NBUF * i + b, b, refill=True, swait=True)
        return carry

    lax.fori_loop(1, (n - NBUF - n % NBUF) // NBUF, body, 0)

    for j in range(n - NBUF - n % NBUF, n):
        visit(j, j % NBUF, refill=j + LEAD < n, swait=True)
    for b in range(NBUF):
        wait_s(b)


@functools.partial(jax.jit, static_argnames=("total_rows",))
def _emb_call(idx1d, weight, total_rows):
    mesh = plsc.VectorSubcoreMesh(core_axis_name="c", subcore_axis_name="s")
    f = pl.kernel(
        _emb_body,
        out_type=jax.ShapeDtypeStruct((total_rows, D), jnp.float32),
        mesh=mesh,
        scratch_types=[
            pltpu.VMEM((total_rows // NW,), jnp.int32),
            pltpu.VMEM((NBUF, CHUNK, D), jnp.float32),
            pltpu.SemaphoreType.DMA((NBUF,)),
            pltpu.SemaphoreType.DMA((NBUF,)),
        ],
    )
    return f(idx1d, weight)


def kernel(token_ids, weight):
    b, s = token_ids.shape
    total = b * s
    idx1d = token_ids.astype(jnp.int32).T.reshape(total)
    out = _emb_call(idx1d, weight, total)
    return out.reshape(s, b, D).transpose(1, 0, 2)
